# Initial kernel scaffold; baseline (speedup 1.0000x reference)
import jax
import jax.numpy as jnp
from jax.experimental import pallas as pl


def kernel(indices, table):
    raise NotImplementedError("write your pallas kernel here")


# speedup vs baseline: 1.1094x; 1.1094x over previous
import functools

import jax
import jax.numpy as jnp
from jax import lax
from jax.experimental import pallas as pl
from jax.experimental.pallas import tpu as pltpu
from jax.experimental.pallas import tpu_sc as plsc


_INFO = plsc.get_sparse_core_info()
_NW = _INFO.num_cores * _INFO.num_subcores
_CHUNK = 1600


@functools.partial(jax.jit, static_argnames=("n_rows", "dim"))
def _gather_rows(flat_idx, table, n_rows, dim):
    per_w = n_rows // _NW
    n_ch = per_w // _CHUNK
    mesh = plsc.VectorSubcoreMesh(core_axis_name="c", subcore_axis_name="s")

    @functools.partial(
        pl.kernel,
        mesh=mesh,
        out_type=jax.ShapeDtypeStruct((n_rows, dim), jnp.float32),
        scratch_types=[
            pltpu.VMEM((_CHUNK,), jnp.int32),
            pltpu.VMEM((_CHUNK,), jnp.int32),
            pltpu.VMEM((_CHUNK, dim), jnp.float32),
            pltpu.VMEM((_CHUNK, dim), jnp.float32),
            pltpu.SemaphoreType.DMA,
            pltpu.SemaphoreType.DMA,
            pltpu.SemaphoreType.DMA,
            pltpu.SemaphoreType.DMA,
            pltpu.SemaphoreType.DMA,
            pltpu.SemaphoreType.DMA,
        ],
        compiler_params=pltpu.CompilerParams(use_tc_tiling_on_sc=False),
    )
    def k(idx_hbm, table_hbm, out_hbm, iv0, iv1, rv0, rv1, si0, si1, sg0, sg1, so0, so1):
        idx_v = (iv0, iv1)
        rows_v = (rv0, rv1)
        si = (si0, si1)
        sg = (sg0, sg1)
        so = (so0, so1)
        wid = lax.axis_index("s") * _INFO.num_cores + lax.axis_index("c")
        base = wid * per_w

        def idx_start(g, s):
            off = pl.multiple_of(base + g * _CHUNK, 8)
            return pltpu.async_copy(idx_hbm.at[pl.ds(off, _CHUNK)], idx_v[s], si[s])

        def gat_start(s):
            return pltpu.async_copy(table_hbm.at[idx_v[s]], rows_v[s], sg[s])

        def out_start(g, s):
            off = pl.multiple_of(base + g * _CHUNK, 8)
            return pltpu.async_copy(rows_v[s], out_hbm.at[pl.ds(off, _CHUNK)], so[s])

        idx_d = [idx_start(0, 0), idx_start(1, 1)]
        gat_d = [None, None]
        sto_d = [None, None]
        idx_d[0].wait()
        gat_d[0] = gat_start(0)
        for g in range(n_ch):
            s = g & 1
            gat_d[s].wait()
            if g + 1 < n_ch:
                idx_d[s ^ 1].wait()
                if sto_d[s ^ 1] is not None:
                    sto_d[s ^ 1].wait()
                gat_d[s ^ 1] = gat_start(s ^ 1)
            sto_d[s] = out_start(g, s)
            if g + 2 < n_ch:
                idx_d[s] = idx_start(g + 2, s)
        if sto_d[(n_ch - 2) & 1] is not None:
            sto_d[(n_ch - 2) & 1].wait()
        sto_d[(n_ch - 1) & 1].wait()

    return k(flat_idx, table)


def kernel(indices, table):
    b, h = indices.shape
    v, d = table.shape
    flat = indices.reshape(b * h).astype(jnp.int32)
    out = _gather_rows(flat, table, b * h, d)
    return out.reshape(b, h, d)


# speedup vs baseline: 1.7746x; 1.5996x over previous
import functools

import jax
import jax.numpy as jnp
from jax import lax
from jax.experimental import pallas as pl
from jax.experimental.pallas import tpu as pltpu
from jax.experimental.pallas import tpu_sc as plsc


_INFO = plsc.get_sparse_core_info()
_NW = _INFO.num_cores * _INFO.num_subcores
_LANES = _INFO.num_lanes
_CHUNK = 1600


@functools.partial(jax.jit, static_argnames=("n_rows",))
def _gather_rows(flat_idx, table16, n_rows):
    per_w = n_rows // _NW
    n_ch = per_w // _CHUNK
    mesh = plsc.VectorSubcoreMesh(core_axis_name="c", subcore_axis_name="s")

    @functools.partial(
        pl.kernel,
        mesh=mesh,
        out_type=jax.ShapeDtypeStruct((2 * n_rows, _LANES), jnp.float32),
        scratch_types=[
            pltpu.VMEM((_CHUNK,), jnp.int32),
            pltpu.VMEM((_CHUNK,), jnp.int32),
            pltpu.VMEM((2 * _CHUNK,), jnp.int32),
            pltpu.VMEM((2 * _CHUNK,), jnp.int32),
            pltpu.VMEM((2 * _CHUNK, _LANES), jnp.float32),
            pltpu.VMEM((2 * _CHUNK, _LANES), jnp.float32),
            pltpu.SemaphoreType.DMA,
            pltpu.SemaphoreType.DMA,
            pltpu.SemaphoreType.DMA,
            pltpu.SemaphoreType.DMA,
            pltpu.SemaphoreType.DMA,
            pltpu.SemaphoreType.DMA,
        ],
        compiler_params=pltpu.CompilerParams(
            use_tc_tiling_on_sc=False, needs_layout_passes=False
        ),
    )
    def k(idx_hbm, table_hbm, out_hbm, iv0, iv1, ev0, ev1, rv0, rv1,
          si0, si1, sg0, sg1, so0, so1):
        idx_v = (iv0, iv1)
        eidx_v = (ev0, ev1)
        rows_v = (rv0, rv1)
        si = (si0, si1)
        sg = (sg0, sg1)
        so = (so0, so1)
        wid = lax.axis_index("s") * _INFO.num_cores + lax.axis_index("c")
        base = wid * per_w
        lanes = lax.iota(jnp.int32, _LANES)

        def idx_start(g, s):
            off = pl.multiple_of(base + g * _CHUNK, 8)
            return pltpu.async_copy(idx_hbm.at[pl.ds(off, _CHUNK)], idx_v[s], si[s])

        def expand(s):
            def body(j, carry):
                off = pl.multiple_of(j * _LANES, 8)
                v = idx_v[s][pl.ds(off, _LANES)]
                v2 = v * 2
                pos = lanes * 2 + 2 * off
                plsc.store_scatter(eidx_v[s], [pos], v2)
                plsc.store_scatter(eidx_v[s], [pos + 1], v2 + 1)
                return carry

            lax.fori_loop(0, _CHUNK // _LANES, body, 0)

        def gat_start(s):
            return pltpu.async_copy(table_hbm.at[eidx_v[s]], rows_v[s], sg[s])

        def out_start(g, s):
            off = pl.multiple_of(2 * (base + g * _CHUNK), 8)
            return pltpu.async_copy(rows_v[s], out_hbm.at[pl.ds(off, 2 * _CHUNK)], so[s])

        idx_d = [idx_start(0, 0), idx_start(1, 1)]
        gat_d = [None, None]
        sto_d = [None, None]
        idx_d[0].wait()
        expand(0)
        gat_d[0] = gat_start(0)
        for g in range(n_ch):
            s = g & 1
            gat_d[s].wait()
            if g + 1 < n_ch:
                idx_d[s ^ 1].wait()
                expand(s ^ 1)
                if sto_d[s ^ 1] is not None:
                    sto_d[s ^ 1].wait()
                gat_d[s ^ 1] = gat_start(s ^ 1)
            sto_d[s] = out_start(g, s)
            if g + 2 < n_ch:
                idx_d[s] = idx_start(g + 2, s)
        if sto_d[(n_ch - 2) & 1] is not None:
            sto_d[(n_ch - 2) & 1].wait()
        sto_d[(n_ch - 1) & 1].wait()

    return k(flat_idx, table16)


def kernel(indices, table):
    b, h = indices.shape
    v, d = table.shape
    flat = indices.reshape(b * h).astype(jnp.int32)
    table16 = table.reshape(v * d // _LANES, _LANES)
    out = _gather_rows(flat, table16, b * h)
    return out.reshape(b, h, d)
